# Initial kernel scaffold; baseline (speedup 1.0000x reference)
import jax
import jax.numpy as jnp
from jax.experimental import pallas as pl


def kernel(seq, adj, W, bias, prelu_a):
    raise NotImplementedError("write your pallas kernel here")


# speedup vs baseline: 14.4781x; 14.4781x over previous
import jax
import jax.numpy as jnp
from jax import lax
from jax.experimental import pallas as pl
from jax.experimental.pallas import tpu as pltpu
from jax.experimental.pallas import tpu_sc as plsc

N = 10000
D = 128
E = 320000

NC = 2
NS = 16
L = 16
NW = NC * NS

CH = 128
CPW = 80
EPW = CH * CPW
EP = NW * EPW
NPAD = 10240
RPW = NPAD // NS
ZCH = RPW // CH
GP = 2
GC = CPW // GP


def _mesh():
    return plsc.VectorSubcoreMesh(
        core_axis_name="c", subcore_axis_name="s", num_cores=NC, num_subcores=NS
    )


def _deg_body(cols_hbm, deg_out, acc, ones_v, stage_v, idx_v):
    cid = lax.axis_index("c")
    sid = lax.axis_index("s")
    wid = sid * NC + cid

    one = jnp.full((L,), 1.0, dtype=jnp.float32)
    zero = jnp.zeros((L,), dtype=jnp.float32)

    def init_ones(i, carry):
        ones_v[i] = one
        return carry

    lax.fori_loop(0, CH, init_ones, 0)

    def init_zero(i, carry):
        stage_v[i] = zero
        return carry

    lax.fori_loop(0, RPW, init_zero, 0)

    pltpu.sync_copy(stage_v, acc.at[pl.ds(sid * RPW, RPW)])
    plsc.subcore_barrier()

    pltpu.sync_copy(cols_hbm.at[wid], idx_v)
    for j in range(CPW):
        pltpu.sync_copy(ones_v, acc.at[idx_v.at[j]], add=True)

    plsc.subcore_barrier()
    pltpu.sync_copy(acc.at[pl.ds(sid * RPW, RPW)], stage_v)
    pltpu.sync_copy(stage_v, deg_out.at[cid, pl.ds(sid * RPW, RPW)])


_deg_call = pl.kernel(
    _deg_body,
    out_type=jax.ShapeDtypeStruct((NC, NPAD, L), jnp.float32),
    mesh=_mesh(),
    scratch_types=[
        pltpu.VMEM_SHARED((NPAD, L), jnp.float32),
        pltpu.VMEM((CH, L), jnp.float32),
        pltpu.VMEM((RPW, L), jnp.float32),
        pltpu.VMEM((CPW, CH), jnp.int32),
    ],
)


def _agg_body(h2_hbm, rows_hbm, cols_hbm, part_out,
              acc, ridx, cidx, buf0, buf1, sem0, sem1):
    cid = lax.axis_index("c")
    sid = lax.axis_index("s")
    wid = sid * NC + cid

    zero = jnp.zeros((L,), dtype=jnp.float32)

    def zr(r, carry):
        for c in range(D // L):
            buf0[r, pl.ds(c * L, L)] = zero
        return carry

    lax.fori_loop(0, CH, zr, 0)
    for k in range(ZCH):
        pltpu.sync_copy(buf0, acc.at[pl.ds(sid * RPW + k * CH, CH)])
    plsc.subcore_barrier()

    bufs = (buf0, buf1)
    sems = (sem0, sem1)
    descs = [None, None]
    for p in range(GP):
        pltpu.sync_copy(rows_hbm.at[wid, pl.ds(p * GC, GC)], ridx)
        pltpu.sync_copy(cols_hbm.at[wid, pl.ds(p * GC, GC)], cidx)
        descs[0] = pltpu.async_copy(h2_hbm.at[ridx.at[0]], buf0, sem0)
        for j in range(GC):
            cur = j & 1
            nxt = cur ^ 1
            if j + 1 < GC:
                descs[nxt] = pltpu.async_copy(
                    h2_hbm.at[ridx.at[j + 1]], bufs[nxt], sems[nxt]
                )
            descs[cur].wait()
            pltpu.sync_copy(bufs[cur], acc.at[cidx.at[j]], add=True)

    plsc.subcore_barrier()
    for k in range(ZCH):
        r0 = sid * RPW + k * CH
        pltpu.sync_copy(acc.at[pl.ds(r0, CH)], buf0)
        pltpu.sync_copy(buf0, part_out.at[cid, pl.ds(r0, CH)])


_agg_call = pl.kernel(
    _agg_body,
    out_type=jax.ShapeDtypeStruct((NC, NPAD, D), jnp.float32),
    mesh=_mesh(),
    scratch_types=[
        pltpu.VMEM_SHARED((NPAD, D), jnp.float32),
        pltpu.VMEM((GC, CH), jnp.int32),
        pltpu.VMEM((GC, CH), jnp.int32),
        pltpu.VMEM((CH, D), jnp.float32),
        pltpu.VMEM((CH, D), jnp.float32),
        pltpu.SemaphoreType.DMA,
        pltpu.SemaphoreType.DMA,
    ],
)


BR = 1000


def _h2_body(seq_ref, w_ref, deg_ref, h2_ref):
    h = jnp.dot(seq_ref[...], w_ref[...], preferred_element_type=jnp.float32)
    deg = 1.0 + deg_ref[0, :, 0:1] + deg_ref[1, :, 0:1]
    h2_ref[...] = h * lax.rsqrt(deg)


_h2_call = pl.pallas_call(
    _h2_body,
    grid=(N // BR,),
    in_specs=[
        pl.BlockSpec((BR, D), lambda i: (i, 0)),
        pl.BlockSpec((D, D), lambda i: (0, 0)),
        pl.BlockSpec((NC, BR, L), lambda i: (0, i, 0)),
    ],
    out_specs=pl.BlockSpec((BR, D), lambda i: (i, 0)),
    out_shape=jax.ShapeDtypeStruct((N, D), jnp.float32),
)


def _out_body(h2_ref, parts_ref, deg_ref, b_ref, a_ref, out_ref):
    s = h2_ref[...] + parts_ref[0] + parts_ref[1]
    deg = 1.0 + deg_ref[0, :, 0:1] + deg_ref[1, :, 0:1]
    o = s * lax.rsqrt(deg) + b_ref[...]
    a = a_ref[0, 0]
    out_ref[...] = jnp.where(o >= 0.0, o, a * o)


_out_call = pl.pallas_call(
    _out_body,
    grid=(N // BR,),
    in_specs=[
        pl.BlockSpec((BR, D), lambda i: (i, 0)),
        pl.BlockSpec((NC, BR, D), lambda i: (0, i, 0)),
        pl.BlockSpec((NC, BR, L), lambda i: (0, i, 0)),
        pl.BlockSpec((1, D), lambda i: (0, 0)),
        pl.BlockSpec((1, 1), lambda i: (0, 0)),
    ],
    out_specs=pl.BlockSpec((BR, D), lambda i: (i, 0)),
    out_shape=jax.ShapeDtypeStruct((N, D), jnp.float32),
)


def kernel(seq, adj, W, bias, prelu_a):
    rows = adj[0]
    cols = adj[1]
    pad = EP - E
    rows_p = jnp.concatenate([rows, jnp.zeros((pad,), dtype=jnp.int32)])
    cols_p = jnp.concatenate([cols, jnp.full((pad,), N, dtype=jnp.int32)])
    rows3 = rows_p.reshape(NW, CPW, CH)
    cols3 = cols_p.reshape(NW, CPW, CH)

    degp = _deg_call(cols3)
    h2 = _h2_call(seq, W, degp)
    parts = _agg_call(h2, rows3, cols3)
    return _out_call(
        h2, parts, degp,
        jnp.reshape(bias, (1, D)),
        jnp.reshape(prelu_a.astype(jnp.float32), (1, 1)),
    )


# speedup vs baseline: 40.2194x; 2.7779x over previous
import jax
import jax.numpy as jnp
from jax import lax
from jax.experimental import pallas as pl
from jax.experimental.pallas import tpu as pltpu
from jax.experimental.pallas import tpu_sc as plsc

N = 10000
D = 128
E = 320000

NC = 2
NS = 16
L = 16
NW = NC * NS

CH = 128
CPW = 80
EPW = CH * CPW
EP = NW * EPW
NPAD = 10240
RPW = NPAD // NS
ZCH = RPW // CH
GP = 2
GC = CPW // GP


def _mesh():
    return plsc.VectorSubcoreMesh(
        core_axis_name="c", subcore_axis_name="s", num_cores=NC, num_subcores=NS
    )


def _deg_body(cols_hbm, deg_out, acc, ones_v, stage_v, idx_v):
    cid = lax.axis_index("c")
    sid = lax.axis_index("s")
    wid = sid * NC + cid

    one = jnp.full((L,), 1.0, dtype=jnp.float32)
    zero = jnp.zeros((L,), dtype=jnp.float32)

    def init_ones(i, carry):
        ones_v[i] = one
        return carry

    lax.fori_loop(0, CH, init_ones, 0)

    def init_zero(i, carry):
        stage_v[i] = zero
        return carry

    lax.fori_loop(0, RPW, init_zero, 0)

    pltpu.sync_copy(stage_v, acc.at[pl.ds(sid * RPW, RPW)])
    plsc.subcore_barrier()

    pltpu.sync_copy(cols_hbm.at[wid], idx_v)
    for j in range(CPW):
        pltpu.sync_copy(ones_v, acc.at[idx_v.at[j]], add=True)

    plsc.subcore_barrier()
    pltpu.sync_copy(acc.at[pl.ds(sid * RPW, RPW)], stage_v)
    pltpu.sync_copy(stage_v, deg_out.at[cid, pl.ds(sid * RPW, RPW)])


_deg_call = pl.kernel(
    _deg_body,
    out_type=jax.ShapeDtypeStruct((NC, NPAD, L), jnp.float32),
    mesh=_mesh(),
    scratch_types=[
        pltpu.VMEM_SHARED((NPAD, L), jnp.float32),
        pltpu.VMEM((CH, L), jnp.float32),
        pltpu.VMEM((RPW, L), jnp.float32),
        pltpu.VMEM((CPW, CH), jnp.int32),
    ],
)


def _agg_body(h2_hbm, rows_hbm, cols_hbm, part_out,
              acc, ridx, cidx, buf0, buf1, sem0, sem1):
    cid = lax.axis_index("c")
    sid = lax.axis_index("s")
    wid = sid * NC + cid

    zero = jnp.zeros((L,), dtype=jnp.float32)

    def zr(r, carry):
        for c in range(D // L):
            buf0[r, pl.ds(c * L, L)] = zero
        return carry

    lax.fori_loop(0, CH, zr, 0)
    for k in range(ZCH):
        pltpu.sync_copy(buf0, acc.at[pl.ds(sid * RPW + k * CH, CH)])
    plsc.subcore_barrier()

    bufs = (buf0, buf1)
    sems = (sem0, sem1)
    descs = [None, None]
    for p in range(GP):
        pltpu.sync_copy(rows_hbm.at[wid, pl.ds(p * GC, GC)], ridx)
        pltpu.sync_copy(cols_hbm.at[wid, pl.ds(p * GC, GC)], cidx)
        descs[0] = pltpu.async_copy(h2_hbm.at[ridx.at[0]], buf0, sem0)
        for j in range(GC):
            cur = j & 1
            nxt = cur ^ 1
            if j + 1 < GC:
                descs[nxt] = pltpu.async_copy(
                    h2_hbm.at[ridx.at[j + 1]], bufs[nxt], sems[nxt]
                )
            descs[cur].wait()
            pltpu.sync_copy(bufs[cur], acc.at[cidx.at[j]], add=True)

    plsc.subcore_barrier()
    for k in range(ZCH):
        r0 = sid * RPW + k * CH
        pltpu.sync_copy(acc.at[pl.ds(r0, CH)], buf0)
        pltpu.sync_copy(buf0, part_out.at[cid, pl.ds(r0, CH)])


_agg_call = pl.kernel(
    _agg_body,
    out_type=jax.ShapeDtypeStruct((NC, NPAD, D), jnp.float32),
    mesh=_mesh(),
    scratch_types=[
        pltpu.VMEM_SHARED((NPAD, D), jnp.float32),
        pltpu.VMEM((GC, CH), jnp.int32),
        pltpu.VMEM((GC, CH), jnp.int32),
        pltpu.VMEM((CH, D), jnp.float32),
        pltpu.VMEM((CH, D), jnp.float32),
        pltpu.SemaphoreType.DMA,
        pltpu.SemaphoreType.DMA,
    ],
)


BR = 1000


def _h2_body(seq_ref, w_ref, deg_ref, h2_ref):
    h = jnp.dot(seq_ref[...], w_ref[...], preferred_element_type=jnp.float32)
    deg = 1.0 + deg_ref[0, :, 0:1] + deg_ref[1, :, 0:1]
    h2_ref[...] = h * lax.rsqrt(deg)


_h2_call = pl.pallas_call(
    _h2_body,
    grid=(N // BR,),
    in_specs=[
        pl.BlockSpec((BR, D), lambda i: (i, 0)),
        pl.BlockSpec((D, D), lambda i: (0, 0)),
        pl.BlockSpec((NC, BR, L), lambda i: (0, i, 0)),
    ],
    out_specs=pl.BlockSpec((BR, D), lambda i: (i, 0)),
    out_shape=jax.ShapeDtypeStruct((N, D), jnp.float32),
)


def _out_body(h2_ref, parts_ref, deg_ref, b_ref, a_ref, out_ref):
    s = h2_ref[...] + parts_ref[0] + parts_ref[1]
    deg = 1.0 + deg_ref[0, :, 0:1] + deg_ref[1, :, 0:1]
    o = s * lax.rsqrt(deg) + b_ref[...]
    a = a_ref[0, 0]
    out_ref[...] = jnp.where(o >= 0.0, o, a * o)


_out_call = pl.pallas_call(
    _out_body,
    grid=(N // BR,),
    in_specs=[
        pl.BlockSpec((BR, D), lambda i: (i, 0)),
        pl.BlockSpec((NC, BR, D), lambda i: (0, i, 0)),
        pl.BlockSpec((NC, BR, L), lambda i: (0, i, 0)),
        pl.BlockSpec((1, D), lambda i: (0, 0)),
        pl.BlockSpec((1, 1), lambda i: (0, 0)),
    ],
    out_specs=pl.BlockSpec((BR, D), lambda i: (i, 0)),
    out_shape=jax.ShapeDtypeStruct((N, D), jnp.float32),
)


def kernel(seq, adj, W, bias, prelu_a):
    rows = adj[0]
    cols = adj[1]
    pad = EP - E
    pk = jnp.arange(pad, dtype=jnp.int32)
    rows_p = jnp.concatenate([rows, pk % N])
    cols_p = jnp.concatenate([cols, N + pk % (NPAD - N)])
    rows3 = rows_p.reshape(NW, CPW, CH)
    cols3 = cols_p.reshape(NW, CPW, CH)

    degp = _deg_call(cols3)
    h2 = _h2_call(seq, W, degp)
    parts = _agg_call(h2, rows3, cols3)
    return _out_call(
        h2, parts, degp,
        jnp.reshape(bias, (1, D)),
        jnp.reshape(prelu_a.astype(jnp.float32), (1, 1)),
    )


# speedup vs baseline: 41.5061x; 1.0320x over previous
import jax
import jax.numpy as jnp
from jax import lax
from jax.experimental import pallas as pl
from jax.experimental.pallas import tpu as pltpu
from jax.experimental.pallas import tpu_sc as plsc

N = 10000
D = 128
E = 320000

NC = 2
NS = 16
L = 16
NW = NC * NS

CH = 128
CPW = 80
EPW = CH * CPW
EP = NW * EPW
NPAD = 10240
RPW = NPAD // NS
ZCH = RPW // CH
GP = 2
GC = CPW // GP


def _mesh():
    return plsc.VectorSubcoreMesh(
        core_axis_name="c", subcore_axis_name="s", num_cores=NC, num_subcores=NS
    )


def _deg_body(cols_hbm, deg_out, acc, ones_v, stage_v, idx_v, sem):
    cid = lax.axis_index("c")
    sid = lax.axis_index("s")
    wid = sid * NC + cid

    one = jnp.full((L,), 1.0, dtype=jnp.float32)
    zero = jnp.zeros((L,), dtype=jnp.float32)

    def init_ones(i, carry):
        ones_v[i] = one
        return carry

    lax.fori_loop(0, CH, init_ones, 0)

    def init_zero(i, carry):
        stage_v[i] = zero
        return carry

    lax.fori_loop(0, RPW, init_zero, 0)

    pltpu.sync_copy(stage_v, acc.at[pl.ds(sid * RPW, RPW)])
    plsc.subcore_barrier()

    pltpu.sync_copy(cols_hbm.at[wid], idx_v)
    descs = [
        pltpu.async_copy(ones_v, acc.at[idx_v.at[j]], sem, add=True)
        for j in range(CPW)
    ]
    for d in descs:
        d.wait()

    plsc.subcore_barrier()
    pltpu.sync_copy(acc.at[pl.ds(sid * RPW, RPW)], stage_v)
    pltpu.sync_copy(stage_v, deg_out.at[cid, pl.ds(sid * RPW, RPW)])


_deg_call = pl.kernel(
    _deg_body,
    out_type=jax.ShapeDtypeStruct((NC, NPAD, L), jnp.float32),
    mesh=_mesh(),
    scratch_types=[
        pltpu.VMEM_SHARED((NPAD, L), jnp.float32),
        pltpu.VMEM((CH, L), jnp.float32),
        pltpu.VMEM((RPW, L), jnp.float32),
        pltpu.VMEM((CPW, CH), jnp.int32),
        pltpu.SemaphoreType.DMA,
    ],
)


def _agg_body(h2_hbm, rows_hbm, cols_hbm, part_out,
              acc, ridx, cidx, buf0, buf1, sem0, sem1):
    cid = lax.axis_index("c")
    sid = lax.axis_index("s")
    wid = sid * NC + cid

    zero = jnp.zeros((L,), dtype=jnp.float32)

    def zr(r, carry):
        for c in range(D // L):
            buf0[r, pl.ds(c * L, L)] = zero
        return carry

    lax.fori_loop(0, CH, zr, 0)
    for k in range(ZCH):
        pltpu.sync_copy(buf0, acc.at[pl.ds(sid * RPW + k * CH, CH)])
    plsc.subcore_barrier()

    bufs = (buf0, buf1)
    sems = (sem0, sem1)
    descs = [None, None]
    for p in range(GP):
        pltpu.sync_copy(rows_hbm.at[wid, pl.ds(p * GC, GC)], ridx)
        pltpu.sync_copy(cols_hbm.at[wid, pl.ds(p * GC, GC)], cidx)
        descs[0] = pltpu.async_copy(h2_hbm.at[ridx.at[0]], buf0, sem0)
        for j in range(GC):
            cur = j & 1
            nxt = cur ^ 1
            if j + 1 < GC:
                descs[nxt] = pltpu.async_copy(
                    h2_hbm.at[ridx.at[j + 1]], bufs[nxt], sems[nxt]
                )
            descs[cur].wait()
            pltpu.sync_copy(bufs[cur], acc.at[cidx.at[j]], add=True)

    plsc.subcore_barrier()
    for k in range(ZCH):
        r0 = sid * RPW + k * CH
        pltpu.sync_copy(acc.at[pl.ds(r0, CH)],
                        part_out.at[cid, pl.ds(r0, CH)])


_agg_call = pl.kernel(
    _agg_body,
    out_type=jax.ShapeDtypeStruct((NC, NPAD, D), jnp.float32),
    mesh=_mesh(),
    scratch_types=[
        pltpu.VMEM_SHARED((NPAD, D), jnp.float32),
        pltpu.VMEM((GC, CH), jnp.int32),
        pltpu.VMEM((GC, CH), jnp.int32),
        pltpu.VMEM((CH, D), jnp.float32),
        pltpu.VMEM((CH, D), jnp.float32),
        pltpu.SemaphoreType.DMA,
        pltpu.SemaphoreType.DMA,
    ],
)


BR = 1000


def _mm_body(seq_ref, w_ref, h_ref):
    h_ref[...] = jnp.dot(
        seq_ref[...], w_ref[...], preferred_element_type=jnp.float32
    )


_mm_call = pl.pallas_call(
    _mm_body,
    grid=(N // BR,),
    in_specs=[
        pl.BlockSpec((BR, D), lambda i: (i, 0)),
        pl.BlockSpec((D, D), lambda i: (0, 0)),
    ],
    out_specs=pl.BlockSpec((BR, D), lambda i: (i, 0)),
    out_shape=jax.ShapeDtypeStruct((N, D), jnp.float32),
)


def _scale_body(h_ref, deg_ref, h2_ref):
    deg = 1.0 + deg_ref[0, :, 0:1] + deg_ref[1, :, 0:1]
    h2_ref[...] = h_ref[...] * lax.rsqrt(deg)


_scale_call = pl.pallas_call(
    _scale_body,
    grid=(N // BR,),
    in_specs=[
        pl.BlockSpec((BR, D), lambda i: (i, 0)),
        pl.BlockSpec((NC, BR, L), lambda i: (0, i, 0)),
    ],
    out_specs=pl.BlockSpec((BR, D), lambda i: (i, 0)),
    out_shape=jax.ShapeDtypeStruct((N, D), jnp.float32),
)


def _out_body(h2_ref, parts_ref, deg_ref, b_ref, a_ref, out_ref):
    s = h2_ref[...] + parts_ref[0] + parts_ref[1]
    deg = 1.0 + deg_ref[0, :, 0:1] + deg_ref[1, :, 0:1]
    o = s * lax.rsqrt(deg) + b_ref[...]
    a = a_ref[0, 0]
    out_ref[...] = jnp.where(o >= 0.0, o, a * o)


_out_call = pl.pallas_call(
    _out_body,
    grid=(N // BR,),
    in_specs=[
        pl.BlockSpec((BR, D), lambda i: (i, 0)),
        pl.BlockSpec((NC, BR, D), lambda i: (0, i, 0)),
        pl.BlockSpec((NC, BR, L), lambda i: (0, i, 0)),
        pl.BlockSpec((1, D), lambda i: (0, 0)),
        pl.BlockSpec((1, 1), lambda i: (0, 0)),
    ],
    out_specs=pl.BlockSpec((BR, D), lambda i: (i, 0)),
    out_shape=jax.ShapeDtypeStruct((N, D), jnp.float32),
)


def kernel(seq, adj, W, bias, prelu_a):
    rows = adj[0]
    cols = adj[1]
    pad = EP - E
    pk = jnp.arange(pad, dtype=jnp.int32)
    rows_p = jnp.concatenate([rows, pk % N])
    cols_p = jnp.concatenate([cols, N + pk % (NPAD - N)])
    rows3 = rows_p.reshape(NW, CPW, CH)
    cols3 = cols_p.reshape(NW, CPW, CH)

    degp = _deg_call(cols3)
    h = _mm_call(seq, W)
    h2 = _scale_call(h, degp)
    parts = _agg_call(h2, rows3, cols3)
    return _out_call(
        h2, parts, degp,
        jnp.reshape(bias, (1, D)),
        jnp.reshape(prelu_a.astype(jnp.float32), (1, 1)),
    )
